# baseline (device time: 19230 ns/iter reference)
import jax
import jax.numpy as jnp
from jax import lax
from jax.experimental import pallas as pl
from jax.experimental.pallas import tpu as pltpu

N_DEV = 4
EPS = 1e-5
N_CHUNKS = 8


def kernel(x, gamma, beta):
    m, n_loc = x.shape
    n_glob = N_DEV * n_loc
    out_dtype = x.dtype
    r = m // N_CHUNKS

    def body(
        x_hbm,
        g_ref,
        b_ref,
        out_hbm,
        xv,
        ov,
        stv,
        comm_ref,
        in_sems,
        out_sems,
        send_sems,
        recv_sems,
    ):
        my = lax.axis_index("i")

        in_copies = []
        for c in range(N_CHUNKS):
            cp = pltpu.make_async_copy(
                x_hbm.at[pl.ds(c * r, r), :],
                xv.at[pl.ds(c * r, r), :],
                in_sems.at[c],
            )
            cp.start()
            in_copies.append(cp)

        barrier_sem = pltpu.get_barrier_semaphore()
        for k in range(1, N_DEV):
            pl.semaphore_signal(
                barrier_sem,
                inc=1,
                device_id=(lax.rem(my + k, N_DEV),),
                device_id_type=pl.DeviceIdType.MESH,
            )
        pl.semaphore_wait(barrier_sem, N_DEV - 1)

        for c in range(N_CHUNKS):
            in_copies[c].wait()
            xc = xv[c * r : (c + 1) * r, :].astype(jnp.float32)
            stv[c * r : (c + 1) * r, 0:1] = jnp.sum(xc, axis=1, keepdims=True)
            stv[c * r : (c + 1) * r, 1:2] = jnp.sum(
                xc * xc, axis=1, keepdims=True
            )

        comm_ref[0, :, :] = jnp.transpose(stv[:, :])

        rdmas = []
        for k in range(1, N_DEV):
            rdma = pltpu.make_async_remote_copy(
                src_ref=comm_ref.at[0],
                dst_ref=comm_ref.at[N_DEV - k],
                send_sem=send_sems.at[k - 1],
                recv_sem=recv_sems.at[N_DEV - k - 1],
                device_id=(lax.rem(my + k, N_DEV),),
                device_id_type=pl.DeviceIdType.MESH,
            )
            rdma.start()
            rdmas.append(rdma)
        for rdma in rdmas:
            rdma.wait()
        acc = (comm_ref[0, :, :] + comm_ref[1, :, :]) + (
            comm_ref[2, :, :] + comm_ref[3, :, :]
        )

        st = jnp.transpose(acc)
        mean = st[:, 0:1] / n_glob
        var = st[:, 1:2] / n_glob - mean * mean
        inv = lax.rsqrt(var + EPS)
        g = g_ref[:, :].astype(jnp.float32)
        b = b_ref[:, :].astype(jnp.float32)

        out_copies = []
        for c in range(N_CHUNKS):
            xc = xv[c * r : (c + 1) * r, :].astype(jnp.float32)
            mc = mean[c * r : (c + 1) * r, :]
            ic = inv[c * r : (c + 1) * r, :]
            ov[c, :, :] = (g * ((xc - mc) * ic) + b).astype(out_dtype)
            cp = pltpu.make_async_copy(
                ov.at[c],
                out_hbm.at[pl.ds(c * r, r), :],
                out_sems.at[c],
            )
            cp.start()
            out_copies.append(cp)
        for cp in out_copies:
            cp.wait()

    return pl.pallas_call(
        body,
        out_shape=jax.ShapeDtypeStruct((m, n_loc), out_dtype),
        in_specs=[
            pl.BlockSpec(memory_space=pl.ANY),
            pl.BlockSpec(memory_space=pltpu.VMEM),
            pl.BlockSpec(memory_space=pltpu.VMEM),
        ],
        out_specs=pl.BlockSpec(memory_space=pl.ANY),
        scratch_shapes=[
            pltpu.VMEM((m, n_loc), x.dtype),
            pltpu.VMEM((N_CHUNKS, m // N_CHUNKS, n_loc), out_dtype),
            pltpu.VMEM((m, 2), jnp.float32),
            pltpu.VMEM((N_DEV, 2, m), jnp.float32),
            pltpu.SemaphoreType.DMA((N_CHUNKS,)),
            pltpu.SemaphoreType.DMA((N_CHUNKS,)),
            pltpu.SemaphoreType.DMA((N_DEV - 1,)),
            pltpu.SemaphoreType.DMA((N_DEV - 1,)),
        ],
        compiler_params=pltpu.CompilerParams(collective_id=0),
    )(x, gamma.reshape(1, n_loc), beta.reshape(1, n_loc))


# device time: 16442 ns/iter; 1.1696x vs baseline; 1.1696x over previous
import os

import jax
import jax.numpy as jnp
from jax import lax
from jax.experimental import pallas as pl
from jax.experimental.pallas import tpu as pltpu

N_DEV = 4
EPS = 1e-5
_ABLATE = os.environ.get("ABL", "")


def kernel(x, gamma, beta):
    m, n_loc = x.shape
    n_glob = N_DEV * n_loc
    out_dtype = x.dtype

    def body(x_ref, g_ref, b_ref, out_ref, comm_ref, send_sems, recv_sems):
        my = lax.axis_index("i")

        if _ABLATE == "copy":
            out_ref[:, :] = x_ref[:, :]
            return

        if _ABLATE in ("", "bar"):
            barrier_sem = pltpu.get_barrier_semaphore()
            for k in range(1, N_DEV):
                pl.semaphore_signal(
                    barrier_sem,
                    inc=1,
                    device_id=(lax.rem(my + k, N_DEV),),
                    device_id_type=pl.DeviceIdType.MESH,
                )

        xf = x_ref[:, :].astype(jnp.float32)
        if _ABLATE == "norm":
            st = jnp.transpose(comm_ref[0, :, :])
            mean = st[:, 0:1] / n_glob
            var = st[:, 1:2] / n_glob - mean * mean
            inv = lax.rsqrt(var + EPS)
            g = g_ref[:, :].astype(jnp.float32)
            b = b_ref[:, :].astype(jnp.float32)
            out_ref[:, :] = (g * ((xf - mean) * inv) + b).astype(out_dtype)
            return

        s = jnp.sum(xf, axis=1, keepdims=True)
        q = jnp.sum(xf * xf, axis=1, keepdims=True)
        comm_ref[0, :, :] = jnp.transpose(
            jnp.concatenate([s, q], axis=1)
        )

        if _ABLATE == "stats":
            out_ref[:, :] = xf.astype(out_dtype)
            return
        if _ABLATE == "nocomm":
            acc = comm_ref[0, :, :] * 4.0
            st = jnp.transpose(acc)
            mean = st[:, 0:1] / n_glob
            var = st[:, 1:2] / n_glob - mean * mean
            inv = lax.rsqrt(var + EPS)
            g = g_ref[:, :].astype(jnp.float32)
            b = b_ref[:, :].astype(jnp.float32)
            out_ref[:, :] = (g * ((xf - mean) * inv) + b).astype(out_dtype)
            return

        pl.semaphore_wait(barrier_sem, N_DEV - 1)
        rdmas = []
        for k in range(1, N_DEV) if _ABLATE != "bar" else []:
            rdma = pltpu.make_async_remote_copy(
                src_ref=comm_ref.at[0],
                dst_ref=comm_ref.at[N_DEV - k],
                send_sem=send_sems.at[k - 1],
                recv_sem=recv_sems.at[N_DEV - k - 1],
                device_id=(lax.rem(my + k, N_DEV),),
                device_id_type=pl.DeviceIdType.MESH,
            )
            rdma.start()
            rdmas.append(rdma)
        for rdma in rdmas:
            rdma.wait_recv()
        if _ABLATE == "bar":
            acc = comm_ref[0, :, :] * 4.0
        else:
            acc = (
                (comm_ref[0, :, :] + comm_ref[1, :, :])
                + (comm_ref[2, :, :] + comm_ref[3, :, :])
            )

        mean_r = acc[0:1, :] * (1.0 / n_glob)
        ex2_r = acc[1:2, :] * (1.0 / n_glob)
        a_r = lax.rsqrt(ex2_r - mean_r * mean_r + EPS)
        c_r = -mean_r * a_r
        act = jnp.transpose(jnp.concatenate([a_r, c_r], axis=0))
        a = act[:, 0:1]
        c = act[:, 1:2]
        g = g_ref[:, :].astype(jnp.float32)
        b = b_ref[:, :].astype(jnp.float32)
        out_ref[:, :] = ((xf * a + c) * g + b).astype(out_dtype)

        for rdma in rdmas:
            rdma.wait_send()

    return pl.pallas_call(
        body,
        out_shape=jax.ShapeDtypeStruct((m, n_loc), out_dtype),
        in_specs=[
            pl.BlockSpec(memory_space=pltpu.VMEM),
            pl.BlockSpec(memory_space=pltpu.VMEM),
            pl.BlockSpec(memory_space=pltpu.VMEM),
        ],
        out_specs=pl.BlockSpec(memory_space=pltpu.VMEM),
        scratch_shapes=[
            pltpu.VMEM((N_DEV, 2, m), jnp.float32),
            pltpu.SemaphoreType.DMA((N_DEV - 1,)),
            pltpu.SemaphoreType.DMA((N_DEV - 1,)),
        ],
        compiler_params=(
            pltpu.CompilerParams(collective_id=0)
            if _ABLATE in ("", "bar")
            else None
        ),
    )(x, gamma.reshape(1, n_loc), beta.reshape(1, n_loc))
